# baseline (speedup 1.0000x reference)
import jax
import jax.numpy as jnp
from jax.experimental import pallas as pl
from jax.experimental.pallas import tpu as pltpu

B, HORIZON, NUM_COV, HID = 8192, 24, 7, 512
INP = NUM_COV + 1
G4 = 4 * HID
KA = NUM_COV + 2
BB = 1024
NB = B // BB
NCHAIN = 2
CB = BB // NCHAIN


def _decoder_kernel(x_ref, h0_ref, c0_ref, y0_ref, wxa_ref, whh_ref,
                    fcw_ref, fcb_ref, out_ref):
    wxa = wxa_ref[...]
    whh = whh_ref[...]
    fcw = fcw_ref[...]
    fcb = fcb_ref[0, 0]
    ones_col = jnp.ones((CB, 1), jnp.float32)

    hs = [h0_ref[q * CB:(q + 1) * CB, :] for q in range(NCHAIN)]
    cs = [c0_ref[q * CB:(q + 1) * CB, :] for q in range(NCHAIN)]
    ys = [y0_ref[q * CB:(q + 1) * CB, :] for q in range(NCHAIN)]

    for t in range(HORIZON):
        for q in range(NCHAIN):
            lo = q * CB
            x_aug = jnp.concatenate(
                [x_ref[lo:lo + CB, t, :], ys[q], ones_col], axis=1)
            gates = (
                jnp.dot(hs[q], whh, preferred_element_type=jnp.float32)
                + jnp.dot(x_aug, wxa, preferred_element_type=jnp.float32)
            )
            ti = jnp.tanh(gates[:, 0 * HID:1 * HID])
            tf = jnp.tanh(gates[:, 1 * HID:2 * HID])
            tg = jnp.tanh(gates[:, 2 * HID:3 * HID])
            to = jnp.tanh(gates[:, 3 * HID:4 * HID])
            c = cs[q]
            c = 0.5 * (tf * c + c + ti * tg + tg)
            cs[q] = c
            tc = jnp.tanh(c)
            hs[q] = to * tc + tc
            logit = jnp.sum(hs[q] * fcw, axis=1, keepdims=True) + fcb
            ys[q] = 0.5 * jnp.tanh(0.5 * logit) + 0.5
            out_ref[lo:lo + CB, t:t + 1] = logit


def kernel(future_x, h_enc, c_enc, y0, W_ih, W_hh, b_ih, b_hh, fc_w, fc_b):
    colscale = jnp.concatenate([
        jnp.full((1, HID), 0.5, jnp.float32),
        jnp.full((1, HID), 0.5, jnp.float32),
        jnp.ones((1, HID), jnp.float32),
        jnp.full((1, HID), 0.5, jnp.float32),
    ], axis=1)
    wxa = jnp.concatenate(
        [W_ih.T, (b_ih + b_hh).reshape(1, G4)], axis=0) * colscale
    whh = W_hh.T * (0.5 * colscale)
    fcw = fc_w * 0.5
    fcb = fc_b.reshape(1, 1)

    out = pl.pallas_call(
        _decoder_kernel,
        grid=(NB,),
        in_specs=[
            pl.BlockSpec((BB, HORIZON, NUM_COV), lambda i: (i, 0, 0)),
            pl.BlockSpec((BB, HID), lambda i: (i, 0)),
            pl.BlockSpec((BB, HID), lambda i: (i, 0)),
            pl.BlockSpec((BB, 1), lambda i: (i, 0)),
            pl.BlockSpec((KA, G4), lambda i: (0, 0)),
            pl.BlockSpec((HID, G4), lambda i: (0, 0)),
            pl.BlockSpec((1, HID), lambda i: (0, 0)),
            pl.BlockSpec((1, 1), lambda i: (0, 0)),
        ],
        out_specs=pl.BlockSpec((BB, HORIZON), lambda i: (i, 0)),
        out_shape=jax.ShapeDtypeStruct((B, HORIZON), jnp.float32),
        compiler_params=pltpu.CompilerParams(
            dimension_semantics=("parallel",),
            vmem_limit_bytes=56 * 1024 * 1024,
        ),
    )(future_x, 2.0 * h_enc, c_enc, y0, wxa, whh, fcw, fcb)
    return out[..., None]
